# Initial kernel scaffold; baseline (speedup 1.0000x reference)
import jax
import jax.numpy as jnp
from jax.experimental import pallas as pl


def kernel(x, edge_index, W1, b1, W2, b2):
    raise NotImplementedError("write your pallas kernel here")


# speedup vs baseline: 19.4511x; 19.4511x over previous
import functools

import jax
import jax.numpy as jnp
from jax import lax
from jax.experimental import pallas as pl
from jax.experimental.pallas import tpu as pltpu
from jax.experimental.pallas import tpu_sc as plsc

N = 10000
E = 320000
D_IN = 128
NHID = 128
LATENT = 64

NC = 2
NS = 16
NW = NC * NS
C = 80
KCH = E // (NW * C)
RPT = N // NS
NP = 10240
RPTP = NP // NS


def _mesh():
    return plsc.VectorSubcoreMesh(core_axis_name="c", subcore_axis_name="s")


def _deg_partials(dst3, zeros_n):
    @functools.partial(
        pl.kernel,
        out_type=jax.ShapeDtypeStruct((NC, 1, NP), jnp.float32),
        mesh=_mesh(),
        scratch_types=[
            pltpu.VMEM((KCH, C), jnp.int32),
            pltpu.VMEM((C,), jnp.float32),
            pltpu.VMEM_SHARED((NP,), jnp.float32),
        ],
    )
    def k(dst3_hbm, zeros_hbm, out_hbm, dst_c, ones_v, deg_sh):
        cid = lax.axis_index("c")
        sid = lax.axis_index("s")
        wid = cid * NS + sid
        rbase = sid * RPTP
        pltpu.sync_copy(zeros_hbm.at[pl.ds(rbase, RPTP)],
                        deg_sh.at[pl.ds(rbase, RPTP)])
        pltpu.sync_copy(dst3_hbm.at[wid], dst_c)
        for j in range(C // 16):
            ones_v[pl.ds(j * 16, 16)] = jnp.full((16,), 1.0, jnp.float32)
        plsc.subcore_barrier()

        def step(kk, carry):
            pltpu.sync_copy(ones_v, deg_sh.at[dst_c.at[kk]], add=True)
            return carry

        lax.fori_loop(0, KCH, step, 0)
        plsc.subcore_barrier()
        pltpu.sync_copy(deg_sh.at[pl.ds(rbase, RPTP)],
                        out_hbm.at[cid, 0, pl.ds(rbase, RPTP)])

    return k(dst3, zeros_n)


def _scatter_partials(table, src3, dst3, zeros_nd, D):
    @functools.partial(
        pl.kernel,
        out_type=jax.ShapeDtypeStruct((NC, NP, D), jnp.float32),
        mesh=_mesh(),
        scratch_types=[
            pltpu.VMEM((KCH, C), jnp.int32),
            pltpu.VMEM((KCH, C), jnp.int32),
            pltpu.VMEM((C, D), jnp.float32),
            pltpu.VMEM_SHARED((NP, D), jnp.float32),
            pltpu.SemaphoreType.DMA,
        ],
    )
    def k(table_hbm, src3_hbm, dst3_hbm, zeros_hbm, out_hbm,
          src_c, dst_c, rows_v, acc_sh, sem):
        cid = lax.axis_index("c")
        sid = lax.axis_index("s")
        wid = cid * NS + sid
        rbase = sid * RPTP
        pltpu.sync_copy(zeros_hbm.at[pl.ds(rbase, RPTP)],
                        acc_sh.at[pl.ds(rbase, RPTP)])
        pltpu.sync_copy(src3_hbm.at[wid], src_c)
        pltpu.sync_copy(dst3_hbm.at[wid], dst_c)
        plsc.subcore_barrier()

        def step(kk, carry):
            pltpu.async_copy(table_hbm.at[src_c.at[kk]], rows_v, sem).wait()
            pltpu.sync_copy(rows_v, acc_sh.at[dst_c.at[kk]], add=True)
            return carry

        lax.fori_loop(0, KCH, step, 0)
        plsc.subcore_barrier()
        pltpu.sync_copy(acc_sh.at[pl.ds(rbase, RPTP)],
                        out_hbm.at[cid, pl.ds(rbase, RPTP)])

    return k(table, src3, dst3, zeros_nd)


BM = 2000
G = N // BM


def _dinv(degp_ref):
    return lax.rsqrt(degp_ref[0] + degp_ref[1] + 1.0)


def _mm1(x, W1, degp):
    def body(x_ref, w_ref, degp_ref, out_ref):
        xw = jnp.dot(x_ref[...], w_ref[...], preferred_element_type=jnp.float32)
        out_ref[...] = xw * _dinv(degp_ref)

    return pl.pallas_call(
        body,
        grid=(G,),
        in_specs=[
            pl.BlockSpec((BM, D_IN), lambda i: (i, 0)),
            pl.BlockSpec((D_IN, NHID), lambda i: (0, 0)),
            pl.BlockSpec((NC, BM, 1), lambda i: (0, i, 0)),
        ],
        out_specs=pl.BlockSpec((BM, NHID), lambda i: (i, 0)),
        out_shape=jax.ShapeDtypeStruct((N, NHID), jnp.float32),
    )(x, W1, degp)


def _mm2(s1p, scaled1, degp, b1r, W2):
    def body(s_ref, sc_ref, degp_ref, b_ref, w_ref, out_ref):
        dinv = _dinv(degp_ref)
        h = jnp.maximum((s_ref[0] + s_ref[1] + sc_ref[...]) * dinv + b_ref[...],
                        0.0)
        out_ref[...] = jnp.dot(h, w_ref[...],
                               preferred_element_type=jnp.float32) * dinv

    return pl.pallas_call(
        body,
        grid=(G,),
        in_specs=[
            pl.BlockSpec((NC, BM, NHID), lambda i: (0, i, 0)),
            pl.BlockSpec((BM, NHID), lambda i: (i, 0)),
            pl.BlockSpec((NC, BM, 1), lambda i: (0, i, 0)),
            pl.BlockSpec((1, NHID), lambda i: (0, 0)),
            pl.BlockSpec((NHID, NHID), lambda i: (0, 0)),
        ],
        out_specs=pl.BlockSpec((BM, NHID), lambda i: (i, 0)),
        out_shape=jax.ShapeDtypeStruct((N, NHID), jnp.float32),
    )(s1p, scaled1, degp, b1r, W2)


def _mm3(s2p, scaled2, degp, b2r):
    def body(s_ref, sc_ref, degp_ref, b_ref, out_ref):
        dinv = _dinv(degp_ref)
        s = s_ref[0, :, :LATENT] + s_ref[1, :, :LATENT] + sc_ref[:, :LATENT]
        out_ref[...] = jnp.maximum(s * dinv + b_ref[...], 0.0)

    return pl.pallas_call(
        body,
        grid=(G,),
        in_specs=[
            pl.BlockSpec((NC, BM, NHID), lambda i: (0, i, 0)),
            pl.BlockSpec((BM, NHID), lambda i: (i, 0)),
            pl.BlockSpec((NC, BM, 1), lambda i: (0, i, 0)),
            pl.BlockSpec((1, LATENT), lambda i: (0, 0)),
        ],
        out_specs=pl.BlockSpec((BM, LATENT), lambda i: (i, 0)),
        out_shape=jax.ShapeDtypeStruct((N, LATENT), jnp.float32),
    )(s2p, scaled2, degp, b2r)


def kernel(x, edge_index, W1, b1, W2, b2):
    src3 = edge_index[0].reshape(NW, KCH, C)
    dst3 = edge_index[1].reshape(NW, KCH, C)
    zeros_n = jnp.zeros((NP,), jnp.float32)
    zeros_h = jnp.zeros((NP, NHID), jnp.float32)
    W2p = jnp.concatenate([W2, jnp.zeros((NHID, NHID - LATENT), jnp.float32)],
                          axis=1)

    degp = _deg_partials(dst3, zeros_n).reshape(NC, NP, 1)
    scaled1 = _mm1(x, W1, degp)
    s1p = _scatter_partials(scaled1, src3, dst3, zeros_h, NHID)
    scaled2 = _mm2(s1p, scaled1, degp, b1.reshape(1, NHID), W2p)
    s2p = _scatter_partials(scaled2, src3, dst3, zeros_h, NHID)
    z = _mm3(s2p, scaled2, degp, b2.reshape(1, LATENT))
    return z


# speedup vs baseline: 28.2066x; 1.4501x over previous
import functools

import jax
import jax.numpy as jnp
from jax import lax
from jax.experimental import pallas as pl
from jax.experimental.pallas import tpu as pltpu
from jax.experimental.pallas import tpu_sc as plsc

N = 10000
E = 320000
D_IN = 128
NHID = 128
LATENT = 64

NC = 2
NS = 16
NW = NC * NS
C = 128
EP = 323584
KCH = EP // (NW * C)
NP = 10240
RPTP = NP // NS


def _mesh():
    return plsc.VectorSubcoreMesh(core_axis_name="c", subcore_axis_name="s")


def _deg_partials(dst3, zeros_n):
    @functools.partial(
        pl.kernel,
        out_type=jax.ShapeDtypeStruct((NC, 1, NP), jnp.float32),
        mesh=_mesh(),
        scratch_types=[
            pltpu.VMEM((KCH, C), jnp.int32),
            pltpu.VMEM((C,), jnp.float32),
            pltpu.VMEM_SHARED((NP,), jnp.float32),
        ],
    )
    def k(dst3_hbm, zeros_hbm, out_hbm, dst_c, ones_v, deg_sh):
        cid = lax.axis_index("c")
        sid = lax.axis_index("s")
        wid = cid * NS + sid
        rbase = sid * RPTP
        pltpu.sync_copy(zeros_hbm.at[pl.ds(rbase, RPTP)],
                        deg_sh.at[pl.ds(rbase, RPTP)])
        pltpu.sync_copy(dst3_hbm.at[wid], dst_c)
        for j in range(C // 16):
            ones_v[pl.ds(j * 16, 16)] = jnp.full((16,), 1.0, jnp.float32)
        plsc.subcore_barrier()

        def step(kk, carry):
            pltpu.sync_copy(ones_v, deg_sh.at[dst_c.at[kk]], add=True)
            return carry

        lax.fori_loop(0, KCH, step, 0)
        plsc.subcore_barrier()
        pltpu.sync_copy(deg_sh.at[pl.ds(rbase, RPTP)],
                        out_hbm.at[cid, 0, pl.ds(rbase, RPTP)])

    return k(dst3, zeros_n)


def _scatter_partials(table, src1, dst1, zeros_nd, D):
    EW = EP // NW

    @functools.partial(
        pl.kernel,
        out_type=jax.ShapeDtypeStruct((NC, NP, D), jnp.float32),
        mesh=_mesh(),
        scratch_types=[
            pltpu.VMEM((2, 1, C), jnp.int32),
            pltpu.VMEM((4, 1, C), jnp.int32),
            pltpu.VMEM((2, C, D), jnp.float32),
            pltpu.VMEM_SHARED((NP, D), jnp.float32),
            pltpu.SemaphoreType.DMA((2,)),
            pltpu.SemaphoreType.DMA((4,)),
            pltpu.SemaphoreType.DMA((2,)),
            pltpu.SemaphoreType.DMA((2,)),
        ],
    )
    def k(table_hbm, src1_hbm, dst1_hbm, zeros_hbm, out_hbm,
          sidx, didx, rows_v, acc_sh, ssem, dsem, gsem, scsem):
        cid = lax.axis_index("c")
        sid = lax.axis_index("s")
        wid = cid * NS + sid
        rbase = sid * RPTP
        ebase = wid * EW

        def iload(kk):
            b2 = lax.rem(kk, 2)
            b4 = lax.rem(kk, 4)
            off = ebase + kk * C
            pltpu.async_copy(src1_hbm.at[pl.ds(off, C)], sidx.at[b2, 0],
                             ssem.at[b2])
            pltpu.async_copy(dst1_hbm.at[pl.ds(off, C)], didx.at[b4, 0],
                             dsem.at[b4])

        def iwait(kk):
            b2 = lax.rem(kk, 2)
            b4 = lax.rem(kk, 4)
            off = ebase + kk * C
            pltpu.make_async_copy(src1_hbm.at[pl.ds(off, C)], sidx.at[b2, 0],
                                  ssem.at[b2]).wait()
            pltpu.make_async_copy(dst1_hbm.at[pl.ds(off, C)], didx.at[b4, 0],
                                  dsem.at[b4]).wait()

        def gstart(kk):
            b2 = lax.rem(kk, 2)
            pltpu.async_copy(table_hbm.at[sidx.at[b2, 0]], rows_v.at[b2],
                             gsem.at[b2])

        def gwait(kk):
            b2 = lax.rem(kk, 2)
            pltpu.make_async_copy(table_hbm.at[sidx.at[b2, 0]], rows_v.at[b2],
                                  gsem.at[b2]).wait()

        def sc_start(kk):
            b2 = lax.rem(kk, 2)
            b4 = lax.rem(kk, 4)
            pltpu.async_copy(rows_v.at[b2], acc_sh.at[didx.at[b4, 0]],
                             scsem.at[b2], add=True)

        def sc_wait(kk):
            b2 = lax.rem(kk, 2)
            b4 = lax.rem(kk, 4)
            pltpu.make_async_copy(rows_v.at[b2], acc_sh.at[didx.at[b4, 0]],
                                  scsem.at[b2]).wait()

        pltpu.sync_copy(zeros_hbm.at[pl.ds(rbase, RPTP)],
                        acc_sh.at[pl.ds(rbase, RPTP)])
        iload(0)
        iload(1)
        iwait(0)
        gstart(0)
        plsc.subcore_barrier()

        def step(kk, carry):
            gwait(kk)

            @pl.when(kk > 0)
            def _():
                sc_wait(kk - 1)

            @pl.when(kk + 1 < KCH)
            def _():
                iwait(kk + 1)
                gstart(kk + 1)

            sc_start(kk)

            @pl.when(kk + 2 < KCH)
            def _():
                iload(kk + 2)

            return carry

        lax.fori_loop(0, KCH, step, 0)
        sc_wait(KCH - 1)
        plsc.subcore_barrier()
        pltpu.sync_copy(acc_sh.at[pl.ds(rbase, RPTP)],
                        out_hbm.at[cid, pl.ds(rbase, RPTP)])

    return k(table, src1, dst1, zeros_nd)


BM = 2000
G = N // BM


def _dinv(degp_ref):
    return lax.rsqrt(degp_ref[0] + degp_ref[1] + 1.0)


def _mm1(x, W1, degp):
    def body(x_ref, w_ref, degp_ref, out_ref):
        xw = jnp.dot(x_ref[...], w_ref[...], preferred_element_type=jnp.float32)
        out_ref[...] = xw * _dinv(degp_ref)

    return pl.pallas_call(
        body,
        grid=(G,),
        in_specs=[
            pl.BlockSpec((BM, D_IN), lambda i: (i, 0)),
            pl.BlockSpec((D_IN, NHID), lambda i: (0, 0)),
            pl.BlockSpec((NC, BM, 1), lambda i: (0, i, 0)),
        ],
        out_specs=pl.BlockSpec((BM, NHID), lambda i: (i, 0)),
        out_shape=jax.ShapeDtypeStruct((N, NHID), jnp.float32),
    )(x, W1, degp)


def _mm2(s1p, scaled1, degp, b1r, W2):
    def body(s_ref, sc_ref, degp_ref, b_ref, w_ref, out_ref):
        dinv = _dinv(degp_ref)
        h = jnp.maximum((s_ref[0] + s_ref[1] + sc_ref[...]) * dinv + b_ref[...],
                        0.0)
        out_ref[...] = jnp.dot(h, w_ref[...],
                               preferred_element_type=jnp.float32) * dinv

    return pl.pallas_call(
        body,
        grid=(G,),
        in_specs=[
            pl.BlockSpec((NC, BM, NHID), lambda i: (0, i, 0)),
            pl.BlockSpec((BM, NHID), lambda i: (i, 0)),
            pl.BlockSpec((NC, BM, 1), lambda i: (0, i, 0)),
            pl.BlockSpec((1, NHID), lambda i: (0, 0)),
            pl.BlockSpec((NHID, NHID), lambda i: (0, 0)),
        ],
        out_specs=pl.BlockSpec((BM, NHID), lambda i: (i, 0)),
        out_shape=jax.ShapeDtypeStruct((N, NHID), jnp.float32),
    )(s1p, scaled1, degp, b1r, W2)


def _mm3(s2p, scaled2, degp, b2r):
    def body(s_ref, sc_ref, degp_ref, b_ref, out_ref):
        dinv = _dinv(degp_ref)
        s = s_ref[0, :, :LATENT] + s_ref[1, :, :LATENT] + sc_ref[:, :LATENT]
        out_ref[...] = jnp.maximum(s * dinv + b_ref[...], 0.0)

    return pl.pallas_call(
        body,
        grid=(G,),
        in_specs=[
            pl.BlockSpec((NC, BM, NHID), lambda i: (0, i, 0)),
            pl.BlockSpec((BM, NHID), lambda i: (i, 0)),
            pl.BlockSpec((NC, BM, 1), lambda i: (0, i, 0)),
            pl.BlockSpec((1, LATENT), lambda i: (0, 0)),
        ],
        out_specs=pl.BlockSpec((BM, LATENT), lambda i: (i, 0)),
        out_shape=jax.ShapeDtypeStruct((N, LATENT), jnp.float32),
    )(s2p, scaled2, degp, b2r)


def kernel(x, edge_index, W1, b1, W2, b2):
    npad = EP - E
    pad_src = (jnp.arange(npad, dtype=jnp.int32) * 131) % N
    pad_dst = N + (jnp.arange(npad, dtype=jnp.int32) % (NP - N))
    src1 = jnp.concatenate([edge_index[0], pad_src])
    dst1 = jnp.concatenate([edge_index[1], pad_dst])
    dst3 = dst1.reshape(NW, KCH, C)
    zeros_n = jnp.zeros((NP,), jnp.float32)
    zeros_h = jnp.zeros((NP, NHID), jnp.float32)
    W2p = jnp.concatenate([W2, jnp.zeros((NHID, NHID - LATENT), jnp.float32)],
                          axis=1)

    degp = _deg_partials(dst3, zeros_n).reshape(NC, NP, 1)
    scaled1 = _mm1(x, W1, degp)
    s1p = _scatter_partials(scaled1, src1, dst1, zeros_h, NHID)
    scaled2 = _mm2(s1p, scaled1, degp, b1.reshape(1, NHID), W2p)
    s2p = _scatter_partials(scaled2, src1, dst1, zeros_h, NHID)
    z = _mm3(s2p, scaled2, degp, b2.reshape(1, LATENT))
    return z
